# Initial kernel scaffold; baseline (speedup 1.0000x reference)
import jax
import jax.numpy as jnp
from jax.experimental import pallas as pl


def kernel(observations, dist_matrix, conv0_W, conv0_b, conv0_g, conv0_beta, conv1_W, conv1_b, conv1_g, conv1_beta, conv2_W, conv2_b, conv2_g, conv2_beta, lin1_W, lin1_b, bn_g, bn_b, lin2_W, lin2_b):
    raise NotImplementedError("write your pallas kernel here")


# speedup vs baseline: 1.0850x; 1.0850x over previous
import jax
import jax.numpy as jnp
from jax import lax
from jax.experimental import pallas as pl

GRIDSIZE = 64
N_NODES = GRIDSIZE * GRIDSIZE
INPUT_DIM = 128
HIDDEN = 128
N_LAYERS = 3
BATCH = 8
BH = BATCH * HIDDEN
MIN_VALUE = -1.0e9

TILE_I = 512
TILE_R = 512


def _layer_body(dm_ref, x_ref, w_ref, b_ref, g_ref, beta_ref, out_ref):
    adj = (dm_ref[...] == 1).astype(jnp.float32)
    agg = lax.dot_general(
        adj, x_ref[...],
        dimension_numbers=(((0,), (0,)), ((), ())),
        preferred_element_type=jnp.float32,
    )
    w = w_ref[...]
    b = b_ref[...]
    g = g_ref[...]
    beta = beta_ref[...]
    for blk in range(BATCH):
        a = agg[:, blk * HIDDEN:(blk + 1) * HIDDEN]
        h = jnp.dot(a, w, preferred_element_type=jnp.float32) + b
        mu = jnp.mean(h, axis=-1, keepdims=True)
        var = jnp.mean((h - mu) ** 2, axis=-1, keepdims=True)
        h = (h - mu) * lax.rsqrt(var + 1e-5) * g + beta
        out_ref[:, blk * HIDDEN:(blk + 1) * HIDDEN] = jnp.maximum(h, 0.0)


def _gin_layer(dm, x, w, b, g, beta):
    n_tiles = N_NODES // TILE_I
    return pl.pallas_call(
        _layer_body,
        grid=(n_tiles,),
        in_specs=[
            pl.BlockSpec((N_NODES, TILE_I), lambda i: (0, i)),
            pl.BlockSpec((N_NODES, BH), lambda i: (0, 0)),
            pl.BlockSpec((HIDDEN, HIDDEN), lambda i: (0, 0)),
            pl.BlockSpec((1, HIDDEN), lambda i: (0, 0)),
            pl.BlockSpec((1, HIDDEN), lambda i: (0, 0)),
            pl.BlockSpec((1, HIDDEN), lambda i: (0, 0)),
        ],
        out_specs=pl.BlockSpec((TILE_I, BH), lambda i: (i, 0)),
        out_shape=jax.ShapeDtypeStruct((N_NODES, BH), jnp.float32),
    )(dm, x, w, b, g, beta)


def _head_body(x0_ref, x1_ref, x2_ref, x3_ref, w1_ref, b1_ref, s_ref, t_ref,
               w2_ref, b2_ref, m_ref, out_ref):
    w1 = w1_ref[...]
    h = jnp.dot(x0_ref[...], w1[0 * HIDDEN:1 * HIDDEN],
                preferred_element_type=jnp.float32)
    h += jnp.dot(x1_ref[...], w1[1 * HIDDEN:2 * HIDDEN],
                 preferred_element_type=jnp.float32)
    h += jnp.dot(x2_ref[...], w1[2 * HIDDEN:3 * HIDDEN],
                 preferred_element_type=jnp.float32)
    h += jnp.dot(x3_ref[...], w1[3 * HIDDEN:4 * HIDDEN],
                 preferred_element_type=jnp.float32)
    h = h + b1_ref[...]
    h = h * s_ref[...] + t_ref[...]
    h = jnp.maximum(h, 0.0)
    o = jnp.dot(h, w2_ref[...], preferred_element_type=jnp.float32)
    o = o + b2_ref[...]
    mask = m_ref[...] != 0.0
    out_ref[...] = jnp.where(mask, o, jnp.float32(MIN_VALUE))


def _head(x0, x1, x2, x3, w1, b1, s, t, w2, b2, msrc):
    rows = BATCH * N_NODES
    n_row_tiles = N_NODES // TILE_R
    grid = (BATCH, n_row_tiles)
    return pl.pallas_call(
        _head_body,
        grid=grid,
        in_specs=[
            pl.BlockSpec((TILE_R, HIDDEN), lambda b, r: (r, b)),
            pl.BlockSpec((TILE_R, HIDDEN), lambda b, r: (r, b)),
            pl.BlockSpec((TILE_R, HIDDEN), lambda b, r: (r, b)),
            pl.BlockSpec((TILE_R, HIDDEN), lambda b, r: (r, b)),
            pl.BlockSpec((4 * HIDDEN, 2 * HIDDEN), lambda b, r: (0, 0)),
            pl.BlockSpec((1, 2 * HIDDEN), lambda b, r: (0, 0)),
            pl.BlockSpec((1, 2 * HIDDEN), lambda b, r: (0, 0)),
            pl.BlockSpec((1, 2 * HIDDEN), lambda b, r: (0, 0)),
            pl.BlockSpec((2 * HIDDEN, 1), lambda b, r: (0, 0)),
            pl.BlockSpec((1, 1), lambda b, r: (0, 0)),
            pl.BlockSpec((TILE_R, 1),
                         lambda b, r: (b * (N_NODES // TILE_R) + r, 0)),
        ],
        out_specs=pl.BlockSpec((TILE_R, 1),
                               lambda b, r: (b * (N_NODES // TILE_R) + r, 0)),
        out_shape=jax.ShapeDtypeStruct((rows, 1), jnp.float32),
    )(x0, x1, x2, x3, w1, b1, s, t, w2, b2, msrc)


def kernel(observations, dist_matrix, conv0_W, conv0_b, conv0_g, conv0_beta,
           conv1_W, conv1_b, conv1_g, conv1_beta, conv2_W, conv2_b, conv2_g,
           conv2_beta, lin1_W, lin1_b, bn_g, bn_b, lin2_W, lin2_b):
    x0 = (observations[:, N_NODES:]
          .reshape(BATCH, N_NODES, INPUT_DIM)
          .transpose(1, 0, 2)
          .reshape(N_NODES, BH))
    msrc = observations[:, :N_NODES].reshape(BATCH * N_NODES, 1)

    params = [
        (conv0_W, conv0_b, conv0_g, conv0_beta),
        (conv1_W, conv1_b, conv1_g, conv1_beta),
        (conv2_W, conv2_b, conv2_g, conv2_beta),
    ]
    xs = [x0]
    x = x0
    for w, b, g, beta in params:
        x = _gin_layer(dist_matrix, x, w, b.reshape(1, HIDDEN),
                       g.reshape(1, HIDDEN), beta.reshape(1, HIDDEN))
        xs.append(x)

    s = (bn_g / jnp.sqrt(1.0 + 1e-5)).reshape(1, 2 * HIDDEN)
    t = bn_b.reshape(1, 2 * HIDDEN)
    out = _head(xs[0], xs[1], xs[2], xs[3], lin1_W,
                lin1_b.reshape(1, 2 * HIDDEN), s, t, lin2_W,
                lin2_b.reshape(1, 1), msrc)
    return out.reshape(BATCH, N_NODES)


# speedup vs baseline: 1.2467x; 1.1490x over previous
import jax
import jax.numpy as jnp
from jax import lax
from jax.experimental import pallas as pl

GRIDSIZE = 64
N_NODES = GRIDSIZE * GRIDSIZE
INPUT_DIM = 128
HIDDEN = 128
BATCH = 8
BH = BATCH * HIDDEN
MIN_VALUE = -1.0e9

TILE_I = 512
N_TILES = N_NODES // TILE_I


def _prep_body(dm_ref, adj_ref):
    adj_ref[...] = (dm_ref[...] == 1).astype(jnp.int8)


def _prep_adj(dm):
    return pl.pallas_call(
        _prep_body,
        grid=(N_TILES,),
        in_specs=[pl.BlockSpec((N_NODES, TILE_I), lambda i: (0, i))],
        out_specs=pl.BlockSpec((N_NODES, TILE_I), lambda i: (0, i)),
        out_shape=jax.ShapeDtypeStruct((N_NODES, N_NODES), jnp.int8),
    )(dm)


def _mlp_ln_relu(agg_blk, w, b, g, beta):
    h = jnp.dot(agg_blk, w, preferred_element_type=jnp.float32) + b
    mu = jnp.mean(h, axis=-1, keepdims=True)
    var = jnp.mean((h - mu) ** 2, axis=-1, keepdims=True)
    h = (h - mu) * lax.rsqrt(var + 1e-5) * g + beta
    return jnp.maximum(h, 0.0)


def _layer_body(adj_ref, x_ref, w_ref, b_ref, g_ref, beta_ref, out_ref):
    adj = adj_ref[...].astype(jnp.bfloat16)
    agg = lax.dot_general(
        adj, x_ref[...],
        dimension_numbers=(((0,), (0,)), ((), ())),
        preferred_element_type=jnp.float32,
    )
    w = w_ref[...]
    b = b_ref[...]
    g = g_ref[...]
    beta = beta_ref[...]
    for blk in range(BATCH):
        y = _mlp_ln_relu(agg[:, blk * HIDDEN:(blk + 1) * HIDDEN], w, b, g, beta)
        out_ref[:, blk * HIDDEN:(blk + 1) * HIDDEN] = y.astype(jnp.bfloat16)


def _gin_layer(adj, x, w, b, g, beta):
    return pl.pallas_call(
        _layer_body,
        grid=(N_TILES,),
        in_specs=[
            pl.BlockSpec((N_NODES, TILE_I), lambda i: (0, i)),
            pl.BlockSpec((N_NODES, BH), lambda i: (0, 0)),
            pl.BlockSpec((HIDDEN, HIDDEN), lambda i: (0, 0)),
            pl.BlockSpec((1, HIDDEN), lambda i: (0, 0)),
            pl.BlockSpec((1, HIDDEN), lambda i: (0, 0)),
            pl.BlockSpec((1, HIDDEN), lambda i: (0, 0)),
        ],
        out_specs=pl.BlockSpec((TILE_I, BH), lambda i: (i, 0)),
        out_shape=jax.ShapeDtypeStruct((N_NODES, BH), jnp.bfloat16),
    )(adj, x, w, b, g, beta)


def _last_body(adj_ref, x2_ref, x0_ref, x1_ref, w_ref, b_ref, g_ref, beta_ref,
               w1_ref, b1_ref, s_ref, t_ref, w2_ref, b2_ref, m_ref, out_ref):
    i = pl.program_id(0)
    adj = adj_ref[...].astype(jnp.bfloat16)
    agg = lax.dot_general(
        adj, x2_ref[...],
        dimension_numbers=(((0,), (0,)), ((), ())),
        preferred_element_type=jnp.float32,
    )
    w = w_ref[...]
    b = b_ref[...]
    g = g_ref[...]
    beta = beta_ref[...]
    w1 = w1_ref[...]
    rows = pl.ds(i * TILE_I, TILE_I)
    for blk in range(BATCH):
        cols = pl.ds(blk * HIDDEN, HIDDEN)
        y3 = _mlp_ln_relu(agg[:, blk * HIDDEN:(blk + 1) * HIDDEN], w, b, g, beta)
        x0 = x0_ref[:, cols].astype(jnp.float32)
        x1 = x1_ref[:, cols].astype(jnp.float32)
        x2 = x2_ref[rows, cols].astype(jnp.float32)
        h = jnp.dot(x0, w1[0 * HIDDEN:1 * HIDDEN],
                    preferred_element_type=jnp.float32)
        h += jnp.dot(x1, w1[1 * HIDDEN:2 * HIDDEN],
                     preferred_element_type=jnp.float32)
        h += jnp.dot(x2, w1[2 * HIDDEN:3 * HIDDEN],
                     preferred_element_type=jnp.float32)
        h += jnp.dot(y3, w1[3 * HIDDEN:4 * HIDDEN],
                     preferred_element_type=jnp.float32)
        h = (h + b1_ref[...]) * s_ref[...] + t_ref[...]
        h = jnp.maximum(h, 0.0)
        o = jnp.dot(h, w2_ref[...], preferred_element_type=jnp.float32)
        o = o + b2_ref[...]
        m = m_ref[:, blk:blk + 1]
        out_ref[:, blk:blk + 1] = jnp.where(m != 0.0, o, jnp.float32(MIN_VALUE))


def _last_layer_and_head(adj, x2, x0, x1, w, b, g, beta, w1, b1, s, t, w2, b2,
                         msrc_t):
    return pl.pallas_call(
        _last_body,
        grid=(N_TILES,),
        in_specs=[
            pl.BlockSpec((N_NODES, TILE_I), lambda i: (0, i)),
            pl.BlockSpec((N_NODES, BH), lambda i: (0, 0)),
            pl.BlockSpec((TILE_I, BH), lambda i: (i, 0)),
            pl.BlockSpec((TILE_I, BH), lambda i: (i, 0)),
            pl.BlockSpec((HIDDEN, HIDDEN), lambda i: (0, 0)),
            pl.BlockSpec((1, HIDDEN), lambda i: (0, 0)),
            pl.BlockSpec((1, HIDDEN), lambda i: (0, 0)),
            pl.BlockSpec((1, HIDDEN), lambda i: (0, 0)),
            pl.BlockSpec((4 * HIDDEN, 2 * HIDDEN), lambda i: (0, 0)),
            pl.BlockSpec((1, 2 * HIDDEN), lambda i: (0, 0)),
            pl.BlockSpec((1, 2 * HIDDEN), lambda i: (0, 0)),
            pl.BlockSpec((1, 2 * HIDDEN), lambda i: (0, 0)),
            pl.BlockSpec((2 * HIDDEN, 1), lambda i: (0, 0)),
            pl.BlockSpec((1, 1), lambda i: (0, 0)),
            pl.BlockSpec((TILE_I, BATCH), lambda i: (i, 0)),
        ],
        out_specs=pl.BlockSpec((TILE_I, BATCH), lambda i: (i, 0)),
        out_shape=jax.ShapeDtypeStruct((N_NODES, BATCH), jnp.float32),
    )(adj, x2, x0, x1, w, b, g, beta, w1, b1, s, t, w2, b2, msrc_t)


def kernel(observations, dist_matrix, conv0_W, conv0_b, conv0_g, conv0_beta,
           conv1_W, conv1_b, conv1_g, conv1_beta, conv2_W, conv2_b, conv2_g,
           conv2_beta, lin1_W, lin1_b, bn_g, bn_b, lin2_W, lin2_b):
    x0 = (observations[:, N_NODES:]
          .reshape(BATCH, N_NODES, INPUT_DIM)
          .transpose(1, 0, 2)
          .reshape(N_NODES, BH)
          .astype(jnp.bfloat16))
    msrc_t = observations[:, :N_NODES].T

    adj = _prep_adj(dist_matrix)
    x1 = _gin_layer(adj, x0, conv0_W, conv0_b.reshape(1, HIDDEN),
                    conv0_g.reshape(1, HIDDEN), conv0_beta.reshape(1, HIDDEN))
    x2 = _gin_layer(adj, x1, conv1_W, conv1_b.reshape(1, HIDDEN),
                    conv1_g.reshape(1, HIDDEN), conv1_beta.reshape(1, HIDDEN))

    s = (bn_g / jnp.sqrt(1.0 + 1e-5)).reshape(1, 2 * HIDDEN)
    t = bn_b.reshape(1, 2 * HIDDEN)
    out_t = _last_layer_and_head(
        adj, x2, x0, x1, conv2_W, conv2_b.reshape(1, HIDDEN),
        conv2_g.reshape(1, HIDDEN), conv2_beta.reshape(1, HIDDEN),
        lin1_W, lin1_b.reshape(1, 2 * HIDDEN), s, t, lin2_W,
        lin2_b.reshape(1, 1), msrc_t)
    return out_t.T


# speedup vs baseline: 1.2638x; 1.0138x over previous
import jax
import jax.numpy as jnp
from jax import lax
from jax.experimental import pallas as pl

GRIDSIZE = 64
N_NODES = GRIDSIZE * GRIDSIZE
INPUT_DIM = 128
HIDDEN = 128
BATCH = 8
BH = BATCH * HIDDEN
MIN_VALUE = -1.0e9

TILE_I = 1024
N_TILES = N_NODES // TILE_I
TILE_P = 512


def _prep_body(dm_ref, adj_ref):
    adj_ref[...] = (dm_ref[...] == 1).astype(jnp.int8)


def _prep_adj(dm):
    return pl.pallas_call(
        _prep_body,
        grid=(N_NODES // TILE_P,),
        in_specs=[pl.BlockSpec((N_NODES, TILE_P), lambda i: (0, i))],
        out_specs=pl.BlockSpec((N_NODES, TILE_P), lambda i: (0, i)),
        out_shape=jax.ShapeDtypeStruct((N_NODES, N_NODES), jnp.int8),
    )(dm)


def _mlp_ln_relu(agg_blk, w, b, g, beta):
    h = jnp.dot(agg_blk, w, preferred_element_type=jnp.float32) + b
    mu = jnp.mean(h, axis=-1, keepdims=True)
    var = jnp.mean((h - mu) ** 2, axis=-1, keepdims=True)
    h = (h - mu) * lax.rsqrt(var + 1e-5) * g + beta
    return jnp.maximum(h, 0.0)


def _layer_body(adj_ref, x_ref, w_ref, b_ref, g_ref, beta_ref, out_ref):
    adj = adj_ref[...].astype(jnp.bfloat16)
    agg = lax.dot_general(
        adj, x_ref[...],
        dimension_numbers=(((0,), (0,)), ((), ())),
        preferred_element_type=jnp.float32,
    )
    w = w_ref[...]
    b = b_ref[...]
    g = g_ref[...]
    beta = beta_ref[...]
    for blk in range(BATCH):
        y = _mlp_ln_relu(agg[:, blk * HIDDEN:(blk + 1) * HIDDEN], w, b, g, beta)
        out_ref[:, blk * HIDDEN:(blk + 1) * HIDDEN] = y.astype(jnp.bfloat16)


def _gin_layer(adj, x, w, b, g, beta):
    return pl.pallas_call(
        _layer_body,
        grid=(N_TILES,),
        in_specs=[
            pl.BlockSpec((N_NODES, TILE_I), lambda i: (0, i)),
            pl.BlockSpec((N_NODES, BH), lambda i: (0, 0)),
            pl.BlockSpec((HIDDEN, HIDDEN), lambda i: (0, 0)),
            pl.BlockSpec((1, HIDDEN), lambda i: (0, 0)),
            pl.BlockSpec((1, HIDDEN), lambda i: (0, 0)),
            pl.BlockSpec((1, HIDDEN), lambda i: (0, 0)),
        ],
        out_specs=pl.BlockSpec((TILE_I, BH), lambda i: (i, 0)),
        out_shape=jax.ShapeDtypeStruct((N_NODES, BH), jnp.bfloat16),
    )(adj, x, w, b, g, beta)


def _last_body(adj_ref, x2_ref, x0_ref, x1_ref, w_ref, b_ref, g_ref, beta_ref,
               w1_ref, b1_ref, s_ref, t_ref, w2_ref, b2_ref, m_ref, out_ref):
    i = pl.program_id(0)
    adj = adj_ref[...].astype(jnp.bfloat16)
    agg = lax.dot_general(
        adj, x2_ref[...],
        dimension_numbers=(((0,), (0,)), ((), ())),
        preferred_element_type=jnp.float32,
    )
    w = w_ref[...]
    b = b_ref[...]
    g = g_ref[...]
    beta = beta_ref[...]
    w1 = w1_ref[...]
    rows = pl.ds(i * TILE_I, TILE_I)
    for blk in range(BATCH):
        cols = pl.ds(blk * HIDDEN, HIDDEN)
        y3 = _mlp_ln_relu(agg[:, blk * HIDDEN:(blk + 1) * HIDDEN], w, b, g, beta)
        x0 = x0_ref[:, cols]
        x1 = x1_ref[:, cols]
        x2 = x2_ref[rows, cols]
        h = jnp.dot(x0, w1[0 * HIDDEN:1 * HIDDEN],
                    preferred_element_type=jnp.float32)
        h += jnp.dot(x1, w1[1 * HIDDEN:2 * HIDDEN],
                     preferred_element_type=jnp.float32)
        h += jnp.dot(x2, w1[2 * HIDDEN:3 * HIDDEN],
                     preferred_element_type=jnp.float32)
        h += jnp.dot(y3.astype(jnp.bfloat16), w1[3 * HIDDEN:4 * HIDDEN],
                     preferred_element_type=jnp.float32)
        h = (h + b1_ref[...]) * s_ref[...] + t_ref[...]
        h = jnp.maximum(h, 0.0)
        o = jnp.dot(h, w2_ref[...], preferred_element_type=jnp.float32)
        o = o + b2_ref[...]
        m = m_ref[:, blk:blk + 1]
        out_ref[:, blk:blk + 1] = jnp.where(m != 0.0, o, jnp.float32(MIN_VALUE))


def _last_layer_and_head(adj, x2, x0, x1, w, b, g, beta, w1, b1, s, t, w2, b2,
                         msrc_t):
    return pl.pallas_call(
        _last_body,
        grid=(N_TILES,),
        in_specs=[
            pl.BlockSpec((N_NODES, TILE_I), lambda i: (0, i)),
            pl.BlockSpec((N_NODES, BH), lambda i: (0, 0)),
            pl.BlockSpec((TILE_I, BH), lambda i: (i, 0)),
            pl.BlockSpec((TILE_I, BH), lambda i: (i, 0)),
            pl.BlockSpec((HIDDEN, HIDDEN), lambda i: (0, 0)),
            pl.BlockSpec((1, HIDDEN), lambda i: (0, 0)),
            pl.BlockSpec((1, HIDDEN), lambda i: (0, 0)),
            pl.BlockSpec((1, HIDDEN), lambda i: (0, 0)),
            pl.BlockSpec((4 * HIDDEN, 2 * HIDDEN), lambda i: (0, 0)),
            pl.BlockSpec((1, 2 * HIDDEN), lambda i: (0, 0)),
            pl.BlockSpec((1, 2 * HIDDEN), lambda i: (0, 0)),
            pl.BlockSpec((1, 2 * HIDDEN), lambda i: (0, 0)),
            pl.BlockSpec((2 * HIDDEN, 1), lambda i: (0, 0)),
            pl.BlockSpec((1, 1), lambda i: (0, 0)),
            pl.BlockSpec((TILE_I, BATCH), lambda i: (i, 0)),
        ],
        out_specs=pl.BlockSpec((TILE_I, BATCH), lambda i: (i, 0)),
        out_shape=jax.ShapeDtypeStruct((N_NODES, BATCH), jnp.float32),
    )(adj, x2, x0, x1, w, b, g, beta, w1, b1, s, t, w2, b2, msrc_t)


def kernel(observations, dist_matrix, conv0_W, conv0_b, conv0_g, conv0_beta,
           conv1_W, conv1_b, conv1_g, conv1_beta, conv2_W, conv2_b, conv2_g,
           conv2_beta, lin1_W, lin1_b, bn_g, bn_b, lin2_W, lin2_b):
    x0 = (observations[:, N_NODES:]
          .reshape(BATCH, N_NODES, INPUT_DIM)
          .transpose(1, 0, 2)
          .reshape(N_NODES, BH)
          .astype(jnp.bfloat16))
    msrc_t = observations[:, :N_NODES].T

    adj = _prep_adj(dist_matrix)
    x1 = _gin_layer(adj, x0, conv0_W, conv0_b.reshape(1, HIDDEN),
                    conv0_g.reshape(1, HIDDEN), conv0_beta.reshape(1, HIDDEN))
    x2 = _gin_layer(adj, x1, conv1_W, conv1_b.reshape(1, HIDDEN),
                    conv1_g.reshape(1, HIDDEN), conv1_beta.reshape(1, HIDDEN))

    s = (bn_g / jnp.sqrt(1.0 + 1e-5)).reshape(1, 2 * HIDDEN)
    t = bn_b.reshape(1, 2 * HIDDEN)
    out_t = _last_layer_and_head(
        adj, x2, x0, x1, conv2_W, conv2_b.reshape(1, HIDDEN),
        conv2_g.reshape(1, HIDDEN), conv2_beta.reshape(1, HIDDEN),
        lin1_W.astype(jnp.bfloat16), lin1_b.reshape(1, 2 * HIDDEN), s, t, lin2_W,
        lin2_b.reshape(1, 1), msrc_t)
    return out_t.T


# speedup vs baseline: 1.3889x; 1.0989x over previous
import jax
import jax.numpy as jnp
from jax import lax
from jax.experimental import pallas as pl

GRIDSIZE = 64
N_NODES = GRIDSIZE * GRIDSIZE
INPUT_DIM = 128
HIDDEN = 128
BATCH = 8
BH = BATCH * HIDDEN
MIN_VALUE = -1.0e9

TILE_F = 512
TILE_I = 1024
N_TILES = N_NODES // TILE_I


def _mlp_ln_relu(agg_blk, w, b, g, beta):
    h = jnp.dot(agg_blk, w, preferred_element_type=jnp.float32) + b
    mu = jnp.mean(h, axis=-1, keepdims=True)
    var = jnp.mean((h - mu) ** 2, axis=-1, keepdims=True)
    h = (h - mu) * lax.rsqrt(var + 1e-5) * g + beta
    return jnp.maximum(h, 0.0)


def _layer_epilogue(agg, w_ref, b_ref, g_ref, beta_ref, out_ref):
    w = w_ref[...]
    b = b_ref[...]
    g = g_ref[...]
    beta = beta_ref[...]
    for blk in range(BATCH):
        y = _mlp_ln_relu(agg[:, blk * HIDDEN:(blk + 1) * HIDDEN], w, b, g, beta)
        out_ref[:, blk * HIDDEN:(blk + 1) * HIDDEN] = y.astype(jnp.bfloat16)


def _first_body(dm_ref, x_ref, w_ref, b_ref, g_ref, beta_ref, adj_ref, out_ref):
    a8 = (dm_ref[...] == 1).astype(jnp.int8)
    adj_ref[...] = a8
    agg = lax.dot_general(
        a8.astype(jnp.bfloat16), x_ref[...],
        dimension_numbers=(((0,), (0,)), ((), ())),
        preferred_element_type=jnp.float32,
    )
    _layer_epilogue(agg, w_ref, b_ref, g_ref, beta_ref, out_ref)


def _first_layer(dm, x, w, b, g, beta):
    return pl.pallas_call(
        _first_body,
        grid=(N_NODES // TILE_F,),
        in_specs=[
            pl.BlockSpec((N_NODES, TILE_F), lambda i: (0, i)),
            pl.BlockSpec((N_NODES, BH), lambda i: (0, 0)),
            pl.BlockSpec((HIDDEN, HIDDEN), lambda i: (0, 0)),
            pl.BlockSpec((1, HIDDEN), lambda i: (0, 0)),
            pl.BlockSpec((1, HIDDEN), lambda i: (0, 0)),
            pl.BlockSpec((1, HIDDEN), lambda i: (0, 0)),
        ],
        out_specs=[
            pl.BlockSpec((N_NODES, TILE_F), lambda i: (0, i)),
            pl.BlockSpec((TILE_F, BH), lambda i: (i, 0)),
        ],
        out_shape=[
            jax.ShapeDtypeStruct((N_NODES, N_NODES), jnp.int8),
            jax.ShapeDtypeStruct((N_NODES, BH), jnp.bfloat16),
        ],
    )(dm, x, w, b, g, beta)


def _mid_body(adj_ref, x_ref, w_ref, b_ref, g_ref, beta_ref, out_ref):
    adj = adj_ref[...].astype(jnp.bfloat16)
    agg = lax.dot_general(
        adj, x_ref[...],
        dimension_numbers=(((0,), (0,)), ((), ())),
        preferred_element_type=jnp.float32,
    )
    _layer_epilogue(agg, w_ref, b_ref, g_ref, beta_ref, out_ref)


def _mid_layer(adj, x, w, b, g, beta):
    return pl.pallas_call(
        _mid_body,
        grid=(N_TILES,),
        in_specs=[
            pl.BlockSpec((N_NODES, TILE_I), lambda i: (0, i)),
            pl.BlockSpec((N_NODES, BH), lambda i: (0, 0)),
            pl.BlockSpec((HIDDEN, HIDDEN), lambda i: (0, 0)),
            pl.BlockSpec((1, HIDDEN), lambda i: (0, 0)),
            pl.BlockSpec((1, HIDDEN), lambda i: (0, 0)),
            pl.BlockSpec((1, HIDDEN), lambda i: (0, 0)),
        ],
        out_specs=pl.BlockSpec((TILE_I, BH), lambda i: (i, 0)),
        out_shape=jax.ShapeDtypeStruct((N_NODES, BH), jnp.bfloat16),
    )(adj, x, w, b, g, beta)


def _last_body(adj_ref, x2_ref, x0_ref, x1_ref, w_ref, b_ref, g_ref, beta_ref,
               w1_ref, b1_ref, s_ref, t_ref, w2_ref, b2_ref, m_ref, out_ref):
    i = pl.program_id(0)
    adj = adj_ref[...].astype(jnp.bfloat16)
    agg = lax.dot_general(
        adj, x2_ref[...],
        dimension_numbers=(((0,), (0,)), ((), ())),
        preferred_element_type=jnp.float32,
    )
    w = w_ref[...]
    b = b_ref[...]
    g = g_ref[...]
    beta = beta_ref[...]
    w1 = w1_ref[...]
    rows = pl.ds(i * TILE_I, TILE_I)
    for blk in range(BATCH):
        cols = pl.ds(blk * HIDDEN, HIDDEN)
        y3 = _mlp_ln_relu(agg[:, blk * HIDDEN:(blk + 1) * HIDDEN], w, b, g, beta)
        x0 = x0_ref[:, cols]
        x1 = x1_ref[:, cols]
        x2 = x2_ref[rows, cols]
        h = jnp.dot(x0, w1[0 * HIDDEN:1 * HIDDEN],
                    preferred_element_type=jnp.float32)
        h += jnp.dot(x1, w1[1 * HIDDEN:2 * HIDDEN],
                     preferred_element_type=jnp.float32)
        h += jnp.dot(x2, w1[2 * HIDDEN:3 * HIDDEN],
                     preferred_element_type=jnp.float32)
        h += jnp.dot(y3.astype(jnp.bfloat16), w1[3 * HIDDEN:4 * HIDDEN],
                     preferred_element_type=jnp.float32)
        h = (h + b1_ref[...]) * s_ref[...] + t_ref[...]
        h = jnp.maximum(h, 0.0)
        o = jnp.dot(h, w2_ref[...], preferred_element_type=jnp.float32)
        o = o + b2_ref[...]
        m = m_ref[:, blk:blk + 1]
        out_ref[:, blk:blk + 1] = jnp.where(m != 0.0, o, jnp.float32(MIN_VALUE))


def _last_layer_and_head(adj, x2, x0, x1, w, b, g, beta, w1, b1, s, t, w2, b2,
                         msrc_t):
    return pl.pallas_call(
        _last_body,
        grid=(N_TILES,),
        in_specs=[
            pl.BlockSpec((N_NODES, TILE_I), lambda i: (0, i)),
            pl.BlockSpec((N_NODES, BH), lambda i: (0, 0)),
            pl.BlockSpec((TILE_I, BH), lambda i: (i, 0)),
            pl.BlockSpec((TILE_I, BH), lambda i: (i, 0)),
            pl.BlockSpec((HIDDEN, HIDDEN), lambda i: (0, 0)),
            pl.BlockSpec((1, HIDDEN), lambda i: (0, 0)),
            pl.BlockSpec((1, HIDDEN), lambda i: (0, 0)),
            pl.BlockSpec((1, HIDDEN), lambda i: (0, 0)),
            pl.BlockSpec((4 * HIDDEN, 2 * HIDDEN), lambda i: (0, 0)),
            pl.BlockSpec((1, 2 * HIDDEN), lambda i: (0, 0)),
            pl.BlockSpec((1, 2 * HIDDEN), lambda i: (0, 0)),
            pl.BlockSpec((1, 2 * HIDDEN), lambda i: (0, 0)),
            pl.BlockSpec((2 * HIDDEN, 1), lambda i: (0, 0)),
            pl.BlockSpec((1, 1), lambda i: (0, 0)),
            pl.BlockSpec((TILE_I, BATCH), lambda i: (i, 0)),
        ],
        out_specs=pl.BlockSpec((TILE_I, BATCH), lambda i: (i, 0)),
        out_shape=jax.ShapeDtypeStruct((N_NODES, BATCH), jnp.float32),
    )(adj, x2, x0, x1, w, b, g, beta, w1, b1, s, t, w2, b2, msrc_t)


def kernel(observations, dist_matrix, conv0_W, conv0_b, conv0_g, conv0_beta,
           conv1_W, conv1_b, conv1_g, conv1_beta, conv2_W, conv2_b, conv2_g,
           conv2_beta, lin1_W, lin1_b, bn_g, bn_b, lin2_W, lin2_b):
    x0 = (observations[:, N_NODES:]
          .reshape(BATCH, N_NODES, INPUT_DIM)
          .transpose(1, 0, 2)
          .reshape(N_NODES, BH)
          .astype(jnp.bfloat16))
    msrc_t = observations[:, :N_NODES].T

    adj, x1 = _first_layer(dist_matrix, x0, conv0_W,
                           conv0_b.reshape(1, HIDDEN),
                           conv0_g.reshape(1, HIDDEN),
                           conv0_beta.reshape(1, HIDDEN))
    x2 = _mid_layer(adj, x1, conv1_W, conv1_b.reshape(1, HIDDEN),
                    conv1_g.reshape(1, HIDDEN), conv1_beta.reshape(1, HIDDEN))

    s = (bn_g / jnp.sqrt(1.0 + 1e-5)).reshape(1, 2 * HIDDEN)
    t = bn_b.reshape(1, 2 * HIDDEN)
    out_t = _last_layer_and_head(
        adj, x2, x0, x1, conv2_W, conv2_b.reshape(1, HIDDEN),
        conv2_g.reshape(1, HIDDEN), conv2_beta.reshape(1, HIDDEN),
        lin1_W.astype(jnp.bfloat16), lin1_b.reshape(1, 2 * HIDDEN), s, t,
        lin2_W, lin2_b.reshape(1, 1), msrc_t)
    return out_t.T
